# baseline (speedup 1.0000x reference)
import jax
import jax.numpy as jnp
from jax import lax
from jax.experimental import pallas as pl
from jax.experimental.pallas import tpu as pltpu
from jax.experimental.pallas import tpu_sc as plsc

_E = 64
_K = 2
_TILE = 4096

_NC = 2
_NS = 16
_NW = _NC * _NS
_CHUNK = 256


def _probs_kernel(x_ref, w_ref, probs_ref):
    x = x_ref[...]
    w = w_ref[...]
    logits = jax.lax.dot_general(
        x, w, (((1,), (1,)), ((), ())), preferred_element_type=jnp.float32
    )
    m = jnp.max(logits, axis=-1, keepdims=True)
    e = jnp.exp(logits - m)
    s = jnp.sum(e, axis=-1, keepdims=True)
    probs_ref[...] = e * (1.0 / s)


def _tc_probs(x_flat, W):
    tokens, d = x_flat.shape
    return pl.pallas_call(
        _probs_kernel,
        grid=(tokens // _TILE,),
        in_specs=[
            pl.BlockSpec((_TILE, d), lambda i: (i, 0)),
            pl.BlockSpec((_E, d), lambda i: (0, 0)),
        ],
        out_specs=pl.BlockSpec((_TILE, _E), lambda i: (i, 0)),
        out_shape=jax.ShapeDtypeStruct((tokens, _E), jnp.float32),
        compiler_params=pltpu.CompilerParams(
            dimension_semantics=("parallel",),
        ),
    )(x_flat, W)


def _sc_topk_body(probs_hbm, vals_hbm, idx_hbm, pbuf, vbuf, ibuf):
    tokens = probs_hbm.shape[0] // _E
    tw = tokens // _NW
    wid = lax.axis_index("s") * _NC + lax.axis_index("c")
    base = wid * tw
    lane = lax.iota(jnp.int32, 16)
    neg = jnp.full((16,), jnp.int32(-(2**31)), jnp.int32)

    def chunk_body(c, carry):
        start = base + c * _CHUNK
        pltpu.sync_copy(probs_hbm.at[pl.ds(start * _E, _CHUNK * _E)], pbuf)

        def group_body(g, carry2):
            rows = lane + g * 16
            pbase = rows * _E
            m1 = neg
            m2 = neg
            for e in range(_E):
                p = plsc.load_gather(pbuf, [pbase + e])
                k = (plsc.bitcast(p, jnp.int32) & jnp.int32(-64)) | jnp.int32(
                    _E - 1 - e
                )
                m2 = jnp.maximum(m2, jnp.minimum(m1, k))
                m1 = jnp.maximum(m1, k)
            i1 = (_E - 1) - (m1 & (_E - 1))
            i2 = (_E - 1) - (m2 & (_E - 1))
            v1 = plsc.load_gather(pbuf, [pbase + i1])
            v2 = plsc.load_gather(pbuf, [pbase + i2])
            obase = rows * _K
            plsc.store_scatter(vbuf, [obase], v1)
            plsc.store_scatter(vbuf, [obase + 1], v2)
            plsc.store_scatter(ibuf, [obase], i1)
            plsc.store_scatter(ibuf, [obase + 1], i2)
            return carry2

        lax.fori_loop(0, _CHUNK // 16, group_body, 0)
        pltpu.sync_copy(vbuf, vals_hbm.at[pl.ds(start * _K, _CHUNK * _K)])
        pltpu.sync_copy(ibuf, idx_hbm.at[pl.ds(start * _K, _CHUNK * _K)])
        return carry

    lax.fori_loop(0, tw // _CHUNK, chunk_body, 0)


def _sc_topk(probs):
    tokens = probs.shape[0]
    vals, idx = pl.kernel(
        _sc_topk_body,
        out_type=[
            jax.ShapeDtypeStruct((tokens * _K,), jnp.float32),
            jax.ShapeDtypeStruct((tokens * _K,), jnp.int32),
        ],
        mesh=plsc.VectorSubcoreMesh(core_axis_name="c", subcore_axis_name="s"),
        compiler_params=pltpu.CompilerParams(needs_layout_passes=False),
        scratch_types=[
            pltpu.VMEM((_CHUNK * _E,), jnp.float32),
            pltpu.VMEM((_CHUNK * _K,), jnp.float32),
            pltpu.VMEM((_CHUNK * _K,), jnp.int32),
        ],
    )(probs.reshape(tokens * _E))
    return vals.reshape(tokens, _K), idx.reshape(tokens, _K)


def kernel(x, W):
    b, n, d = x.shape
    x_flat = x.reshape(b * n, d)
    probs = _tc_probs(x_flat, W)
    vals, idx = _sc_topk(probs)
    return (probs, vals, idx)


# speedup vs baseline: 1.1079x; 1.1079x over previous
import jax
import jax.numpy as jnp
from jax import lax
from jax.experimental import pallas as pl
from jax.experimental.pallas import tpu as pltpu
from jax.experimental.pallas import tpu_sc as plsc

_E = 64
_K = 2
_TILE = 4096

_NC = 2
_NS = 16
_NW = _NC * _NS
_CHUNK = 256


def _probs_kernel(x_ref, w_ref, probs_ref):
    x = x_ref[...]
    w = w_ref[...]
    logits = jax.lax.dot_general(
        x, w, (((1,), (1,)), ((), ())), preferred_element_type=jnp.float32
    )
    m = jnp.max(logits, axis=-1, keepdims=True)
    e = jnp.exp(logits - m)
    s = jnp.sum(e, axis=-1, keepdims=True)
    probs_ref[...] = e * (1.0 / s)


def _tc_probs(x_flat, W):
    tokens, d = x_flat.shape
    return pl.pallas_call(
        _probs_kernel,
        grid=(tokens // _TILE,),
        in_specs=[
            pl.BlockSpec((_TILE, d), lambda i: (i, 0)),
            pl.BlockSpec((_E, d), lambda i: (0, 0)),
        ],
        out_specs=pl.BlockSpec((_TILE, _E), lambda i: (i, 0)),
        out_shape=jax.ShapeDtypeStruct((tokens, _E), jnp.float32),
        compiler_params=pltpu.CompilerParams(
            dimension_semantics=("parallel",),
        ),
    )(x_flat, W)


def _sc_topk_body(probs_hbm, vals_hbm, idx_hbm, pbuf0, pbuf1, vbuf, ibuf,
                  sem0, sem1):
    tokens = probs_hbm.shape[0] // _E
    tw = tokens // _NW
    wid = lax.axis_index("s") * _NC + lax.axis_index("c")
    base = wid * tw
    lane = lax.iota(jnp.int32, 16)
    neg = jnp.full((16,), jnp.int32(-(2**31)), jnp.int32)
    n_chunks = tw // _CHUNK
    pbufs = (pbuf0, pbuf1)
    sems = (sem0, sem1)

    def start_dma(c):
        start = base + c * _CHUNK
        return pltpu.async_copy(
            probs_hbm.at[pl.ds(start * _E, _CHUNK * _E)],
            pbufs[c % 2],
            sems[c % 2],
        )

    cur = start_dma(0)
    for c in range(n_chunks):
        nxt = start_dma(c + 1) if c + 1 < n_chunks else None
        cur.wait()
        pbuf = pbufs[c % 2]

        def group_body(g, carry2):
            rows = lane + g * 16
            pbase = rows * _E
            m1 = neg
            m2 = neg
            for i in range(_E):
                evec = (lane + i) & (_E - 1)
                p = plsc.load_gather(pbuf, [pbase + evec])
                k = (plsc.bitcast(p, jnp.int32) & jnp.int32(-64)) | (
                    evec ^ (_E - 1)
                )
                m2 = jnp.maximum(m2, jnp.minimum(m1, k))
                m1 = jnp.maximum(m1, k)
            i1 = (_E - 1) - (m1 & (_E - 1))
            i2 = (_E - 1) - (m2 & (_E - 1))
            v1 = plsc.load_gather(pbuf, [pbase + i1])
            v2 = plsc.load_gather(pbuf, [pbase + i2])
            obase = rows * _K
            plsc.store_scatter(vbuf, [obase], v1)
            plsc.store_scatter(vbuf, [obase + 1], v2)
            plsc.store_scatter(ibuf, [obase], i1)
            plsc.store_scatter(ibuf, [obase + 1], i2)
            return carry2

        lax.fori_loop(0, _CHUNK // 16, group_body, 0)
        start = base + c * _CHUNK
        pltpu.sync_copy(vbuf, vals_hbm.at[pl.ds(start * _K, _CHUNK * _K)])
        pltpu.sync_copy(ibuf, idx_hbm.at[pl.ds(start * _K, _CHUNK * _K)])
        cur = nxt


def _sc_topk(probs):
    tokens = probs.shape[0]
    vals, idx = pl.kernel(
        _sc_topk_body,
        out_type=[
            jax.ShapeDtypeStruct((tokens * _K,), jnp.float32),
            jax.ShapeDtypeStruct((tokens * _K,), jnp.int32),
        ],
        mesh=plsc.VectorSubcoreMesh(core_axis_name="c", subcore_axis_name="s"),
        compiler_params=pltpu.CompilerParams(needs_layout_passes=False),
        scratch_types=[
            pltpu.VMEM((_CHUNK * _E,), jnp.float32),
            pltpu.VMEM((_CHUNK * _E,), jnp.float32),
            pltpu.VMEM((_CHUNK * _K,), jnp.float32),
            pltpu.VMEM((_CHUNK * _K,), jnp.int32),
            pltpu.SemaphoreType.DMA,
            pltpu.SemaphoreType.DMA,
        ],
    )(probs.reshape(tokens * _E))
    return vals.reshape(tokens, _K), idx.reshape(tokens, _K)


def kernel(x, W):
    b, n, d = x.shape
    x_flat = x.reshape(b * n, d)
    probs = _tc_probs(x_flat, W)
    vals, idx = _sc_topk(probs)
    return (probs, vals, idx)


# speedup vs baseline: 1.1972x; 1.0806x over previous
import jax
import jax.numpy as jnp
from jax import lax
from jax.experimental import pallas as pl
from jax.experimental.pallas import tpu as pltpu
from jax.experimental.pallas import tpu_sc as plsc

_E = 64
_K = 2
_TILE = 4096

_NC = 2
_NS = 16
_NW = _NC * _NS
_CHUNK = 256


def _probs_kernel(x_ref, w_ref, probs_ref):
    x = x_ref[...]
    w = w_ref[...]
    logits = jax.lax.dot_general(
        x, w, (((1,), (1,)), ((), ())), preferred_element_type=jnp.float32
    )
    m = jnp.max(logits, axis=-1, keepdims=True)
    e = jnp.exp(logits - m)
    s = jnp.sum(e, axis=-1, keepdims=True)
    probs_ref[...] = e * (1.0 / s)


def _tc_probs(x_flat, W):
    tokens, d = x_flat.shape
    return pl.pallas_call(
        _probs_kernel,
        grid=(tokens // _TILE,),
        in_specs=[
            pl.BlockSpec((_TILE, d), lambda i: (i, 0)),
            pl.BlockSpec((_E, d), lambda i: (0, 0)),
        ],
        out_specs=pl.BlockSpec((_TILE, _E), lambda i: (i, 0)),
        out_shape=jax.ShapeDtypeStruct((tokens, _E), jnp.float32),
        compiler_params=pltpu.CompilerParams(
            dimension_semantics=("parallel",),
        ),
    )(x_flat, W)


def _sc_topk_body(probs_hbm, vals_hbm, idx_hbm, pbuf0, pbuf1, vbuf, ibuf,
                  sem0, sem1):
    tokens = probs_hbm.shape[0]
    tw = tokens // _NW
    wid = lax.axis_index("s") * _NC + lax.axis_index("c")
    base = wid * tw
    lane = lax.iota(jnp.int32, 16)
    neg = jnp.full((16,), jnp.int32(-(2**31)), jnp.int32)
    n_chunks = tw // _CHUNK
    pbufs = (pbuf0, pbuf1)
    sems = (sem0, sem1)

    def start_dma(c):
        start = base + c * _CHUNK
        return pltpu.async_copy(
            probs_hbm.at[pl.ds(start, _CHUNK)],
            pbufs[c % 2],
            sems[c % 2],
        )

    cur = start_dma(0)
    for c in range(n_chunks):
        nxt = start_dma(c + 1) if c + 1 < n_chunks else None
        cur.wait()
        pbuf = pbufs[c % 2]

        def group_body(g, carry2):
            rows = lane + g * 16
            m1 = neg
            m2 = neg
            for i in range(_E):
                evec = (lane + i) & (_E - 1)
                p = plsc.load_gather(pbuf, [rows, evec])
                k = (plsc.bitcast(p, jnp.int32) & jnp.int32(-64)) | (
                    evec ^ (_E - 1)
                )
                m2 = jnp.maximum(m2, jnp.minimum(m1, k))
                m1 = jnp.maximum(m1, k)
            i1 = (_E - 1) - (m1 & (_E - 1))
            i2 = (_E - 1) - (m2 & (_E - 1))
            v1 = plsc.load_gather(pbuf, [rows, i1])
            v2 = plsc.load_gather(pbuf, [rows, i2])
            obase = rows * _K
            plsc.store_scatter(vbuf, [obase], v1)
            plsc.store_scatter(vbuf, [obase + 1], v2)
            plsc.store_scatter(ibuf, [obase], i1)
            plsc.store_scatter(ibuf, [obase + 1], i2)
            return carry2

        lax.fori_loop(0, _CHUNK // 16, group_body, 0)
        start = base + c * _CHUNK
        pltpu.sync_copy(vbuf, vals_hbm.at[pl.ds(start * _K, _CHUNK * _K)])
        pltpu.sync_copy(ibuf, idx_hbm.at[pl.ds(start * _K, _CHUNK * _K)])
        cur = nxt


def _sc_topk(probs):
    tokens = probs.shape[0]
    vals, idx = pl.kernel(
        _sc_topk_body,
        out_type=[
            jax.ShapeDtypeStruct((tokens * _K,), jnp.float32),
            jax.ShapeDtypeStruct((tokens * _K,), jnp.int32),
        ],
        mesh=plsc.VectorSubcoreMesh(core_axis_name="c", subcore_axis_name="s"),
        compiler_params=pltpu.CompilerParams(
            needs_layout_passes=False, use_tc_tiling_on_sc=True
        ),
        scratch_types=[
            pltpu.VMEM((_CHUNK, _E), jnp.float32),
            pltpu.VMEM((_CHUNK, _E), jnp.float32),
            pltpu.VMEM((_CHUNK * _K,), jnp.float32),
            pltpu.VMEM((_CHUNK * _K,), jnp.int32),
            pltpu.SemaphoreType.DMA,
            pltpu.SemaphoreType.DMA,
        ],
    )(probs)
    return vals.reshape(tokens, _K), idx.reshape(tokens, _K)


def kernel(x, W):
    b, n, d = x.shape
    x_flat = x.reshape(b * n, d)
    probs = _tc_probs(x_flat, W)
    vals, idx = _sc_topk(probs)
    return (probs, vals, idx)


# speedup vs baseline: 1.2059x; 1.0073x over previous
import jax
import jax.numpy as jnp
from jax import lax
from jax.experimental import pallas as pl
from jax.experimental.pallas import tpu as pltpu
from jax.experimental.pallas import tpu_sc as plsc

_E = 64
_K = 2
_TILE = 4096

_NC = 2
_NS = 16
_NW = _NC * _NS
_CHUNK = 256


def _probs_kernel(x_ref, w_ref, probs_ref):
    x = x_ref[...]
    w = w_ref[...]
    logits = jax.lax.dot_general(
        x, w, (((1,), (1,)), ((), ())), preferred_element_type=jnp.float32
    )
    m = jnp.max(logits, axis=-1, keepdims=True)
    e = jnp.exp(logits - m)
    s = jnp.sum(e, axis=-1, keepdims=True)
    probs_ref[...] = e * (1.0 / s)


def _tc_probs(x_flat, W):
    tokens, d = x_flat.shape
    return pl.pallas_call(
        _probs_kernel,
        grid=(tokens // _TILE,),
        in_specs=[
            pl.BlockSpec((_TILE, d), lambda i: (i, 0)),
            pl.BlockSpec((_E, d), lambda i: (0, 0)),
        ],
        out_specs=pl.BlockSpec((_TILE, _E), lambda i: (i, 0)),
        out_shape=jax.ShapeDtypeStruct((tokens, _E), jnp.float32),
        compiler_params=pltpu.CompilerParams(
            dimension_semantics=("parallel",),
        ),
    )(x_flat, W)


def _sc_topk_body(probs_hbm, vals_hbm, idx_hbm, pbuf0, pbuf1, vbuf, ibuf,
                  sem0, sem1):
    tokens = probs_hbm.shape[0]
    tw = tokens // _NW
    wid = lax.axis_index("s") * _NC + lax.axis_index("c")
    base = wid * tw
    lane = lax.iota(jnp.int32, 16)
    neg = jnp.full((16,), jnp.int32(-(2**31)), jnp.int32)
    n_chunks = tw // _CHUNK
    pbufs = (pbuf0, pbuf1)
    sems = (sem0, sem1)

    def start_dma(c):
        start = base + c * _CHUNK
        return pltpu.async_copy(
            probs_hbm.at[pl.ds(start, _CHUNK)],
            pbufs[c % 2],
            sems[c % 2],
        )

    cur = start_dma(0)
    for c in range(n_chunks):
        nxt = start_dma(c + 1) if c + 1 < n_chunks else None
        cur.wait()
        pbuf = pbufs[c % 2]

        def group_body(g, carry2):
            rows = lane + g * 16
            m1 = neg
            m2 = neg
            for i in range(_E):
                evec = (lane + i) & (_E - 1)
                p = plsc.load_gather(pbuf, [rows, evec])
                k = (plsc.bitcast(p, jnp.int32) & jnp.int32(-64)) | (
                    evec ^ (_E - 1)
                )
                m2 = jnp.maximum(m2, jnp.minimum(m1, k))
                m1 = jnp.maximum(m1, k)
            i1 = (_E - 1) - (m1 & (_E - 1))
            i2 = (_E - 1) - (m2 & (_E - 1))
            v1 = plsc.bitcast(m1 & jnp.int32(-64), jnp.float32)
            v2 = plsc.bitcast(m2 & jnp.int32(-64), jnp.float32)
            obase = rows * _K
            plsc.store_scatter(vbuf, [obase], v1)
            plsc.store_scatter(vbuf, [obase + 1], v2)
            plsc.store_scatter(ibuf, [obase], i1)
            plsc.store_scatter(ibuf, [obase + 1], i2)
            return carry2

        lax.fori_loop(0, _CHUNK // 16, group_body, 0)
        start = base + c * _CHUNK
        pltpu.sync_copy(vbuf, vals_hbm.at[pl.ds(start * _K, _CHUNK * _K)])
        pltpu.sync_copy(ibuf, idx_hbm.at[pl.ds(start * _K, _CHUNK * _K)])
        cur = nxt


def _sc_topk(probs):
    tokens = probs.shape[0]
    vals, idx = pl.kernel(
        _sc_topk_body,
        out_type=[
            jax.ShapeDtypeStruct((tokens * _K,), jnp.float32),
            jax.ShapeDtypeStruct((tokens * _K,), jnp.int32),
        ],
        mesh=plsc.VectorSubcoreMesh(core_axis_name="c", subcore_axis_name="s"),
        compiler_params=pltpu.CompilerParams(
            needs_layout_passes=False, use_tc_tiling_on_sc=True
        ),
        scratch_types=[
            pltpu.VMEM((_CHUNK, _E), jnp.float32),
            pltpu.VMEM((_CHUNK, _E), jnp.float32),
            pltpu.VMEM((_CHUNK * _K,), jnp.float32),
            pltpu.VMEM((_CHUNK * _K,), jnp.int32),
            pltpu.SemaphoreType.DMA,
            pltpu.SemaphoreType.DMA,
        ],
    )(probs)
    return vals.reshape(tokens, _K), idx.reshape(tokens, _K)


def kernel(x, W):
    b, n, d = x.shape
    x_flat = x.reshape(b * n, d)
    probs = _tc_probs(x_flat, W)
    vals, idx = _sc_topk(probs)
    return (probs, vals, idx)


# speedup vs baseline: 1.2329x; 1.0224x over previous
import jax
import jax.numpy as jnp
from jax import lax
from jax.experimental import pallas as pl
from jax.experimental.pallas import tpu as pltpu
from jax.experimental.pallas import tpu_sc as plsc

_E = 64
_K = 2
_TILE = 4096

_NC = 2
_NS = 16
_NW = _NC * _NS
_CHUNK = 256


def _probs_kernel(x_ref, w_ref, probs_ref):
    x = x_ref[...]
    w = w_ref[...]
    logits = jax.lax.dot_general(
        x, w, (((1,), (1,)), ((), ())), preferred_element_type=jnp.float32
    )
    m = jnp.max(logits, axis=-1, keepdims=True)
    e = jnp.exp(logits - m)
    s = jnp.sum(e, axis=-1, keepdims=True)
    probs_ref[...] = e * (1.0 / s)


def _tc_probs(x_flat, W):
    tokens, d = x_flat.shape
    return pl.pallas_call(
        _probs_kernel,
        grid=(tokens // _TILE,),
        in_specs=[
            pl.BlockSpec((_TILE, d), lambda i: (i, 0)),
            pl.BlockSpec((_E, d), lambda i: (0, 0)),
        ],
        out_specs=pl.BlockSpec((_TILE, _E), lambda i: (i, 0)),
        out_shape=jax.ShapeDtypeStruct((tokens, _E), jnp.float32),
        compiler_params=pltpu.CompilerParams(
            dimension_semantics=("parallel",),
        ),
    )(x_flat, W)


def _sc_topk_body(probs_hbm, vals_hbm, idx_hbm, pbuf0, pbuf1, vbuf, ibuf,
                  sem0, sem1):
    tokens = probs_hbm.shape[0]
    tw = tokens // _NW
    wid = lax.axis_index("s") * _NC + lax.axis_index("c")
    base = wid * tw
    lane = lax.iota(jnp.int32, 16)
    neg = jnp.full((16,), jnp.int32(-(2**31)), jnp.int32)
    n_chunks = tw // _CHUNK
    pbufs = (pbuf0, pbuf1)
    sems = (sem0, sem1)

    def start_dma(c):
        start = base + c * _CHUNK
        return pltpu.async_copy(
            probs_hbm.at[pl.ds(start, _CHUNK)],
            pbufs[c % 2],
            sems[c % 2],
        )

    cur = start_dma(0)
    for c in range(n_chunks):
        nxt = start_dma(c + 1) if c + 1 < n_chunks else None
        cur.wait()
        pbuf = pbufs[c % 2]

        @plsc.parallel_loop(0, _CHUNK // 16, unroll=2)
        def group_body(g):
            rows = lane + g * 16
            m1 = neg
            m2 = neg
            for i in range(_E):
                evec = (lane + i) & (_E - 1)
                p = plsc.load_gather(pbuf, [rows, evec])
                k = (plsc.bitcast(p, jnp.int32) & jnp.int32(-64)) | (
                    evec ^ (_E - 1)
                )
                m2 = jnp.maximum(m2, jnp.minimum(m1, k))
                m1 = jnp.maximum(m1, k)
            i1 = (_E - 1) - (m1 & (_E - 1))
            i2 = (_E - 1) - (m2 & (_E - 1))
            v1 = plsc.bitcast(m1 & jnp.int32(-64), jnp.float32)
            v2 = plsc.bitcast(m2 & jnp.int32(-64), jnp.float32)
            obase = rows * _K
            plsc.store_scatter(vbuf, [obase], v1)
            plsc.store_scatter(vbuf, [obase + 1], v2)
            plsc.store_scatter(ibuf, [obase], i1)
            plsc.store_scatter(ibuf, [obase + 1], i2)

        start = base + c * _CHUNK
        pltpu.sync_copy(vbuf, vals_hbm.at[pl.ds(start * _K, _CHUNK * _K)])
        pltpu.sync_copy(ibuf, idx_hbm.at[pl.ds(start * _K, _CHUNK * _K)])
        cur = nxt


def _sc_topk(probs):
    tokens = probs.shape[0]
    vals, idx = pl.kernel(
        _sc_topk_body,
        out_type=[
            jax.ShapeDtypeStruct((tokens * _K,), jnp.float32),
            jax.ShapeDtypeStruct((tokens * _K,), jnp.int32),
        ],
        mesh=plsc.VectorSubcoreMesh(core_axis_name="c", subcore_axis_name="s"),
        compiler_params=pltpu.CompilerParams(
            needs_layout_passes=False, use_tc_tiling_on_sc=True
        ),
        scratch_types=[
            pltpu.VMEM((_CHUNK, _E), jnp.float32),
            pltpu.VMEM((_CHUNK, _E), jnp.float32),
            pltpu.VMEM((_CHUNK * _K,), jnp.float32),
            pltpu.VMEM((_CHUNK * _K,), jnp.int32),
            pltpu.SemaphoreType.DMA,
            pltpu.SemaphoreType.DMA,
        ],
    )(probs)
    return vals.reshape(tokens, _K), idx.reshape(tokens, _K)


def kernel(x, W):
    b, n, d = x.shape
    x_flat = x.reshape(b * n, d)
    probs = _tc_probs(x_flat, W)
    vals, idx = _sc_topk(probs)
    return (probs, vals, idx)


# speedup vs baseline: 1.2437x; 1.0088x over previous
import jax
import jax.numpy as jnp
from jax import lax
from jax.experimental import pallas as pl
from jax.experimental.pallas import tpu as pltpu
from jax.experimental.pallas import tpu_sc as plsc

_E = 64
_K = 2
_TILE = 4096

_NC = 2
_NS = 16
_NW = _NC * _NS
_CHUNK = 256


def _probs_kernel(x_ref, w_ref, probs_ref):
    x = x_ref[...]
    w = w_ref[...]
    logits = jax.lax.dot_general(
        x, w, (((1,), (1,)), ((), ())), preferred_element_type=jnp.float32
    )
    m = jnp.max(logits, axis=-1, keepdims=True)
    e = jnp.exp(logits - m)
    s = jnp.sum(e, axis=-1, keepdims=True)
    probs_ref[...] = e * (1.0 / s)


def _tc_probs(x_flat, W):
    tokens, d = x_flat.shape
    return pl.pallas_call(
        _probs_kernel,
        grid=(tokens // _TILE,),
        in_specs=[
            pl.BlockSpec((_TILE, d), lambda i: (i, 0)),
            pl.BlockSpec((_E, d), lambda i: (0, 0)),
        ],
        out_specs=pl.BlockSpec((_TILE, _E), lambda i: (i, 0)),
        out_shape=jax.ShapeDtypeStruct((tokens, _E), jnp.float32),
        compiler_params=pltpu.CompilerParams(
            dimension_semantics=("parallel",),
        ),
    )(x_flat, W)


def _sc_topk_body(probs_hbm, vals_hbm, idx_hbm, pbuf0, pbuf1, vbuf, ibuf,
                  sem0, sem1):
    tokens = probs_hbm.shape[0]
    tw = tokens // _NW
    wid = lax.axis_index("s") * _NC + lax.axis_index("c")
    base = wid * tw
    lane = lax.iota(jnp.int32, 16)
    neg = jnp.full((16,), jnp.int32(-(2**31)), jnp.int32)
    n_chunks = tw // _CHUNK
    pbufs = (pbuf0, pbuf1)
    sems = (sem0, sem1)

    def start_dma(c):
        start = base + c * _CHUNK
        return pltpu.async_copy(
            probs_hbm.at[pl.ds(start, _CHUNK)],
            pbufs[c % 2],
            sems[c % 2],
        )

    cur = start_dma(0)
    for c in range(n_chunks):
        nxt = start_dma(c + 1) if c + 1 < n_chunks else None
        cur.wait()
        pbuf = pbufs[c % 2]

        @plsc.parallel_loop(0, _CHUNK // 16, unroll=2)
        def group_body(g):
            rows = lane + g * 16
            m1v = neg
            m1i = neg
            m2v = neg
            m2i = neg
            for i in range(_E):
                evec = (lane + i) & (_E - 1)
                p = plsc.bitcast(plsc.load_gather(pbuf, [rows, evec]), jnp.int32)
                c1 = p > m1v
                t = jnp.where(c1, m1v, p)
                ti = jnp.where(c1, m1i, evec)
                c2 = t > m2v
                m2v = jnp.where(c2, t, m2v)
                m2i = jnp.where(c2, ti, m2i)
                m1v = jnp.where(c1, p, m1v)
                m1i = jnp.where(c1, evec, m1i)
            obase = rows * _K
            plsc.store_scatter(vbuf, [obase], plsc.bitcast(m1v, jnp.float32))
            plsc.store_scatter(vbuf, [obase + 1], plsc.bitcast(m2v, jnp.float32))
            plsc.store_scatter(ibuf, [obase], m1i)
            plsc.store_scatter(ibuf, [obase + 1], m2i)

        start = base + c * _CHUNK
        pltpu.sync_copy(vbuf, vals_hbm.at[pl.ds(start * _K, _CHUNK * _K)])
        pltpu.sync_copy(ibuf, idx_hbm.at[pl.ds(start * _K, _CHUNK * _K)])
        cur = nxt


def _sc_topk(probs):
    tokens = probs.shape[0]
    vals, idx = pl.kernel(
        _sc_topk_body,
        out_type=[
            jax.ShapeDtypeStruct((tokens * _K,), jnp.float32),
            jax.ShapeDtypeStruct((tokens * _K,), jnp.int32),
        ],
        mesh=plsc.VectorSubcoreMesh(core_axis_name="c", subcore_axis_name="s"),
        compiler_params=pltpu.CompilerParams(
            needs_layout_passes=False, use_tc_tiling_on_sc=True
        ),
        scratch_types=[
            pltpu.VMEM((_CHUNK, _E), jnp.float32),
            pltpu.VMEM((_CHUNK, _E), jnp.float32),
            pltpu.VMEM((_CHUNK * _K,), jnp.float32),
            pltpu.VMEM((_CHUNK * _K,), jnp.int32),
            pltpu.SemaphoreType.DMA,
            pltpu.SemaphoreType.DMA,
        ],
    )(probs)
    return vals.reshape(tokens, _K), idx.reshape(tokens, _K)


def kernel(x, W):
    b, n, d = x.shape
    x_flat = x.reshape(b * n, d)
    probs = _tc_probs(x_flat, W)
    vals, idx = _sc_topk(probs)
    return (probs, vals, idx)
